# baseline (speedup 1.0000x reference)
import jax
import jax.numpy as jnp
from jax.experimental import pallas as pl
from jax.experimental.pallas import tpu as pltpu

_NC = 3
_NSET = 2


def _plane_kernel(na_ref, col_ref, row_ref, out_hbm, scratch, sems):
    b = pl.program_id(0)
    nb = pl.num_programs(0)
    sset = jax.lax.rem(b, _NSET)
    na = na_ref[b]

    n = col_ref.shape[2]
    jio = jax.lax.broadcasted_iota(jnp.int32, (n, n), 0)
    kio = jax.lax.broadcasted_iota(jnp.int32, (n, n), 1)
    mask = (jio < na) & (kio < na)

    half = n // 2

    @pl.when(b >= _NSET)
    def _wait_prev():
        for c in range(_NC):
            for h in range(2):
                pltpu.make_async_copy(
                    scratch.at[sset, c, pl.ds(h * half, half)],
                    out_hbm.at[b - _NSET, c, pl.ds(h * half, half)],
                    sems.at[sset, c, h]).wait()

    for c in range(_NC):
        col = col_ref[0, c]
        row = row_ref[0, c]
        scratch[sset, c] = jnp.where(mask, col - row, jnp.float32(0.0))
        for h in range(2):
            pltpu.make_async_copy(
                scratch.at[sset, c, pl.ds(h * half, half)],
                out_hbm.at[b, c, pl.ds(h * half, half)],
                sems.at[sset, c, h]).start()

    @pl.when(b == nb - 1)
    def _drain():
        for s in range(_NSET):
            prev = nb - _NSET + s
            for c in range(_NC):
                for h in range(2):
                    pltpu.make_async_copy(
                        scratch.at[jax.lax.rem(jnp.int32(prev), _NSET), c,
                                   pl.ds(h * half, half)],
                        out_hbm.at[prev, c, pl.ds(h * half, half)],
                        sems.at[jax.lax.rem(jnp.int32(prev), _NSET), c, h]).wait()


def kernel(coords, num_atoms):
    bsz, flat = coords.shape
    maxa = flat // 3
    xt = coords.reshape(bsz, maxa, 3).transpose(0, 2, 1)
    xcol = xt.reshape(bsz, 3, maxa, 1)
    xrow = xt.reshape(bsz, 3, 1, maxa)
    na = num_atoms.astype(jnp.int32)
    out = pl.pallas_call(
        _plane_kernel,
        grid_spec=pltpu.PrefetchScalarGridSpec(
            num_scalar_prefetch=1,
            grid=(bsz,),
            in_specs=[
                pl.BlockSpec((1, _NC, maxa, 1), lambda b, na_ref: (b, 0, 0, 0)),
                pl.BlockSpec((1, _NC, 1, maxa), lambda b, na_ref: (b, 0, 0, 0)),
            ],
            out_specs=pl.BlockSpec(memory_space=pl.ANY),
            scratch_shapes=[
                pltpu.VMEM((_NSET, _NC, maxa, maxa), jnp.float32),
                pltpu.SemaphoreType.DMA((_NSET, _NC, 2)),
            ],
        ),
        out_shape=jax.ShapeDtypeStruct((bsz, _NC, maxa, maxa), jnp.float32),
    )(na, xcol, xrow)
    return out.transpose(0, 2, 3, 1)


# speedup vs baseline: 1.6268x; 1.6268x over previous
import jax
import jax.numpy as jnp
from jax.experimental import pallas as pl
from jax.experimental.pallas import tpu as pltpu

_NC = 3
_NSET = 2


def _plane_kernel(na_ref, row_ref, out_hbm, scratch, sems):
    b = pl.program_id(0)
    nb = pl.num_programs(0)
    sset = jax.lax.rem(b, _NSET)
    na = na_ref[b]

    c3t = row_ref[0, :, 0, :]
    c3 = c3t.T

    n = c3.shape[0]
    jio = jax.lax.broadcasted_iota(jnp.int32, (n, n), 0)
    kio = jax.lax.broadcasted_iota(jnp.int32, (n, n), 1)
    mask = (jio < na) & (kio < na)
    half = n // 2

    @pl.when(b >= _NSET)
    def _wait_prev():
        for c in range(_NC):
            for h in range(2):
                pltpu.make_async_copy(
                    scratch.at[sset, c, pl.ds(h * half, half)],
                    out_hbm.at[b - _NSET, c, pl.ds(h * half, half)],
                    sems.at[sset, c, h]).wait()

    for c in range(_NC):
        col = c3[:, c:c + 1]
        row = c3t[c:c + 1, :]
        scratch[sset, c] = jnp.where(mask, col - row, jnp.float32(0.0))
        for h in range(2):
            pltpu.make_async_copy(
                scratch.at[sset, c, pl.ds(h * half, half)],
                out_hbm.at[b, c, pl.ds(h * half, half)],
                sems.at[sset, c, h]).start()

    @pl.when(b == nb - 1)
    def _drain():
        for s in range(_NSET):
            prev = nb - _NSET + s
            for c in range(_NC):
                for h in range(2):
                    pltpu.make_async_copy(
                        scratch.at[jax.lax.rem(jnp.int32(prev), _NSET), c,
                                   pl.ds(h * half, half)],
                        out_hbm.at[prev, c, pl.ds(h * half, half)],
                        sems.at[jax.lax.rem(jnp.int32(prev), _NSET), c, h]).wait()


def kernel(coords, num_atoms):
    bsz, flat = coords.shape
    maxa = flat // 3
    na = num_atoms.astype(jnp.int32)
    out = pl.pallas_call(
        _plane_kernel,
        grid_spec=pltpu.PrefetchScalarGridSpec(
            num_scalar_prefetch=1,
            grid=(bsz,),
            in_specs=[
                pl.BlockSpec((1, _NC, 1, maxa), lambda b, na_ref: (b, 0, 0, 0)),
            ],
            out_specs=pl.BlockSpec(memory_space=pl.ANY),
            scratch_shapes=[
                pltpu.VMEM((_NSET, _NC, maxa, maxa), jnp.float32),
                pltpu.SemaphoreType.DMA((_NSET, _NC, 2)),
            ],
        ),
        out_shape=jax.ShapeDtypeStruct((bsz, _NC, maxa, maxa), jnp.float32),
    )(na, coords.reshape(bsz, maxa, 3).transpose(0, 2, 1).reshape(
        bsz, _NC, 1, maxa))
    return out.transpose(0, 2, 3, 1)
